# baseline (speedup 1.0000x reference)
import functools

import jax
import jax.numpy as jnp
from jax import lax
from jax.experimental import pallas as pl
from jax.experimental.pallas import tpu as pltpu
from jax.experimental.pallas import tpu_sc as plsc

_LANES = 16


def _sc_gather(t, table_a, table_c):
    B = t.shape[0]
    info = plsc.get_sparse_core_info()
    nc, ns = info.num_cores, info.num_subcores
    nw = nc * ns
    b_per_w = B // nw

    mesh = plsc.VectorSubcoreMesh(core_axis_name="c", subcore_axis_name="s")

    @functools.partial(
        pl.kernel,
        mesh=mesh,
        out_type=(
            jax.ShapeDtypeStruct((B,), jnp.float32),
            jax.ShapeDtypeStruct((B,), jnp.float32),
        ),
        scratch_types=[
            pltpu.VMEM((b_per_w,), jnp.int32),
            pltpu.VMEM((b_per_w,), jnp.float32),
            pltpu.VMEM((b_per_w,), jnp.float32),
            pltpu.SemaphoreType.DMA,
        ],
    )
    def gather_kernel(t_hbm, a_hbm, c_hbm, oa_hbm, oc_hbm,
                      idx_v, oa_v, oc_v, sem):
        wid = lax.axis_index("s") * nc + lax.axis_index("c")
        base = wid * b_per_w
        pltpu.sync_copy(t_hbm.at[pl.ds(base, b_per_w)], idx_v)
        cp_a = pltpu.async_copy(a_hbm.at[idx_v], oa_v, sem)
        cp_c = pltpu.async_copy(c_hbm.at[idx_v], oc_v, sem)
        cp_a.wait()
        cp_c.wait()
        pltpu.sync_copy(oa_v, oa_hbm.at[pl.ds(base, b_per_w)])
        pltpu.sync_copy(oc_v, oc_hbm.at[pl.ds(base, b_per_w)])

    return gather_kernel(t, table_a, table_c)


def _blend_body(a_ref, c_ref, x_ref, n_ref, o_ref):
    o_ref[...] = a_ref[...] * x_ref[...] + c_ref[...] * n_ref[...]


def _tc_blend(x2d, n2d, a2, c2, bblk):
    B, F = x2d.shape
    return pl.pallas_call(
        _blend_body,
        grid=(B // bblk,),
        in_specs=[
            pl.BlockSpec((bblk, 1), lambda i: (i, 0)),
            pl.BlockSpec((bblk, 1), lambda i: (i, 0)),
            pl.BlockSpec((bblk, F), lambda i: (i, 0)),
            pl.BlockSpec((bblk, F), lambda i: (i, 0)),
        ],
        out_specs=pl.BlockSpec((bblk, F), lambda i: (i, 0)),
        out_shape=jax.ShapeDtypeStruct((B, F), jnp.float32),
    )(a2, c2, x2d, n2d)


def kernel(x_0, t, noise, sqrt_alphas_cumprod, sqrt_one_minus_alphas_cumprod):
    B, L, D = x_0.shape
    t32 = t.astype(jnp.int32)
    a_g, c_g = _sc_gather(t32, sqrt_alphas_cumprod,
                          sqrt_one_minus_alphas_cumprod)
    F = L * D
    out2d = _tc_blend(x_0.reshape(B, F), noise.reshape(B, F),
                      a_g.reshape(B, 1), c_g.reshape(B, 1), 64)
    return out2d.reshape(B, L, D), noise


# speedup vs baseline: 1.0011x; 1.0011x over previous
import functools

import jax
import jax.numpy as jnp
from jax import lax
from jax.experimental import pallas as pl
from jax.experimental.pallas import tpu as pltpu
from jax.experimental.pallas import tpu_sc as plsc

_LANES = 16


def _sc_gather(t, table_a, table_c):
    B = t.shape[0]
    info = plsc.get_sparse_core_info()
    nc, ns = info.num_cores, info.num_subcores
    nw = nc * ns
    b_per_w = B // nw

    mesh = plsc.VectorSubcoreMesh(core_axis_name="c", subcore_axis_name="s")

    @functools.partial(
        pl.kernel,
        mesh=mesh,
        out_type=(
            jax.ShapeDtypeStruct((B,), jnp.float32),
            jax.ShapeDtypeStruct((B,), jnp.float32),
        ),
        scratch_types=[
            pltpu.VMEM((b_per_w,), jnp.int32),
            pltpu.VMEM((b_per_w,), jnp.float32),
            pltpu.VMEM((b_per_w,), jnp.float32),
            pltpu.SemaphoreType.DMA,
        ],
    )
    def gather_kernel(t_hbm, a_hbm, c_hbm, oa_hbm, oc_hbm,
                      idx_v, oa_v, oc_v, sem):
        wid = lax.axis_index("s") * nc + lax.axis_index("c")
        base = wid * b_per_w
        pltpu.sync_copy(t_hbm.at[pl.ds(base, b_per_w)], idx_v)
        cp_a = pltpu.async_copy(a_hbm.at[idx_v], oa_v, sem)
        cp_c = pltpu.async_copy(c_hbm.at[idx_v], oc_v, sem)
        cp_a.wait()
        cp_c.wait()
        pltpu.sync_copy(oa_v, oa_hbm.at[pl.ds(base, b_per_w)])
        pltpu.sync_copy(oc_v, oc_hbm.at[pl.ds(base, b_per_w)])

    return gather_kernel(t, table_a, table_c)


def _blend_body(a_ref, c_ref, x_ref, n_ref, o_ref):
    o_ref[...] = a_ref[:, 0:1] * x_ref[...] + c_ref[:, 0:1] * n_ref[...]


def _tc_blend(x2d, n2d, a2, c2, bblk):
    B, F = x2d.shape
    return pl.pallas_call(
        _blend_body,
        grid=(B // bblk,),
        in_specs=[
            pl.BlockSpec((bblk, 128), lambda i: (i, 0)),
            pl.BlockSpec((bblk, 128), lambda i: (i, 0)),
            pl.BlockSpec((bblk, F), lambda i: (i, 0)),
            pl.BlockSpec((bblk, F), lambda i: (i, 0)),
        ],
        out_specs=pl.BlockSpec((bblk, F), lambda i: (i, 0)),
        out_shape=jax.ShapeDtypeStruct((B, F), jnp.float32),
    )(a2, c2, x2d, n2d)


def kernel(x_0, t, noise, sqrt_alphas_cumprod, sqrt_one_minus_alphas_cumprod):
    B, L, D = x_0.shape
    t32 = t.astype(jnp.int32)
    a_g, c_g = _sc_gather(t32, sqrt_alphas_cumprod,
                          sqrt_one_minus_alphas_cumprod)
    F = L * D
    a_b = jnp.broadcast_to(a_g[:, None], (B, 128))
    c_b = jnp.broadcast_to(c_g[:, None], (B, 128))
    out2d = _tc_blend(x_0.reshape(B, F), noise.reshape(B, F),
                      a_b, c_b, 64)
    return out2d.reshape(B, L, D), noise
